# baseline (speedup 1.0000x reference)
import functools

import jax
import jax.numpy as jnp
from jax import lax
from jax.experimental import pallas as pl
from jax.experimental.pallas import tpu as pltpu
from jax.experimental.pallas import tpu_sc as plsc

N_EMB = 1000
ROWS = 4096 * 20
NC, NS, L = 2, 16, 16
ROWS_PER_T = ROWS // (NC * NS)
Z = 1_280_000
SC_FLOATS = ROWS // NC * N_EMB
Z_PER_T = SC_FLOATS // (NS * Z)
K = 128
NK = ROWS_PER_T // K


def _one_hot_sc(idx_hbm, zeros_hbm):
    mesh = plsc.VectorSubcoreMesh(
        core_axis_name="c", subcore_axis_name="s", num_cores=NC, num_subcores=NS
    )

    @functools.partial(
        pl.kernel,
        out_type=jax.ShapeDtypeStruct((ROWS * N_EMB,), jnp.float32),
        mesh=mesh,
        scratch_types=[
            pltpu.VMEM_SHARED((Z,), jnp.float32),
            pltpu.VMEM((ROWS_PER_T,), jnp.int32),
            pltpu.VMEM((NK, K), jnp.int32),
            pltpu.VMEM((K,), jnp.float32),
            pltpu.SemaphoreType.DMA,
            pltpu.SemaphoreType.DMA,
        ],
        compiler_params=pltpu.CompilerParams(use_tc_tiling_on_sc=True),
    )
    def body(idx_ref, zeros_ref, out_ref, zblk, idx_v, pos_v, ones_b, sem_z, sem_s):
        cid = lax.axis_index("c")
        sid = lax.axis_index("s")
        tile_row0 = (cid * NS + sid) * ROWS_PER_T

        @pl.when(sid == 0)
        def _():
            pltpu.sync_copy(zeros_ref, zblk)

        plsc.subcore_barrier()

        sc_base = cid * SC_FLOATS
        for k in range(Z_PER_T):
            dst = out_ref.at[pl.ds(sc_base + (sid * Z_PER_T + k) * Z, Z)]
            pltpu.async_copy(zblk, dst, sem_z)

        pltpu.sync_copy(idx_ref.at[pl.ds(tile_row0, ROWS_PER_T)], idx_v)
        lane = lax.iota(jnp.int32, L)
        ones_v = jnp.full((L,), 1.0, jnp.float32)
        for q in range(K // L):
            ones_b[pl.ds(q * L, L)] = ones_v

        def fill_pos(n, carry):
            for q in range(K // L):
                g = n * K + q * L
                jl = g // 256
                il = g - jl * 256
                it = sid * 2 + il // 128
                sterm = (
                    (cid * 10 + jl) * (N_EMB * 4096)
                    + it * 1024
                    + (il % 128)
                )
                k16 = idx_v[pl.ds(g, L)]
                pos_v[n, pl.ds(q * L, L)] = (
                    sterm + (k16 >> 3) * 32768 + (k16 & 7) * 128 + lane
                )
            return carry

        lax.fori_loop(0, NK, fill_pos, 0)

        for k in range(Z_PER_T):
            pltpu.make_async_copy(zblk, out_ref.at[pl.ds(0, Z)], sem_z).wait()
        plsc.subcore_barrier()

        for j in range(NK):
            pltpu.async_copy(ones_b, out_ref.at[pos_v.at[j]], sem_s)
        for j in range(NK):
            pltpu.make_async_copy(ones_b, out_ref.at[pos_v.at[0]], sem_s).wait()

    return body(idx_hbm, zeros_hbm)


@jax.jit
def kernel(inputs):
    idx_t = inputs.astype(jnp.int32).T
    idx_perm = jnp.transpose(
        idx_t.reshape(NC, 10, NS, 256), (0, 2, 1, 3)
    ).reshape(-1)
    zeros = jnp.zeros((Z,), jnp.float32)
    flat = _one_hot_sc(idx_perm, zeros)
    out5 = flat.reshape(20, N_EMB // 8, 4096 // 128, 8, 128)
    out3 = jnp.transpose(out5, (0, 1, 3, 2, 4)).reshape(20, N_EMB, 4096)
    return jnp.transpose(out3, (2, 0, 1))


# speedup vs baseline: 1.0168x; 1.0168x over previous
import functools

import jax
import jax.numpy as jnp
from jax import lax
from jax.experimental import pallas as pl
from jax.experimental.pallas import tpu as pltpu
from jax.experimental.pallas import tpu_sc as plsc

N_EMB = 1000
ROWS = 4096 * 20
NC, NS, L = 2, 16, 16
ROWS_PER_T = ROWS // (NC * NS)
Z = 640_000
SC_FLOATS = ROWS // NC * N_EMB
Z_PER_T = SC_FLOATS // (NS * Z)
K = 128
NK = ROWS_PER_T // K


def _one_hot_sc(idx_hbm, zeros_hbm):
    mesh = plsc.VectorSubcoreMesh(
        core_axis_name="c", subcore_axis_name="s", num_cores=NC, num_subcores=NS
    )

    @functools.partial(
        pl.kernel,
        out_type=jax.ShapeDtypeStruct((ROWS * N_EMB,), jnp.float32),
        mesh=mesh,
        scratch_types=[
            pltpu.VMEM_SHARED((Z,), jnp.float32),
            pltpu.VMEM((ROWS_PER_T,), jnp.int32),
            pltpu.VMEM((NK, K), jnp.int32),
            pltpu.VMEM((K,), jnp.float32),
            pltpu.SemaphoreType.DMA,
            pltpu.SemaphoreType.DMA,
        ],
        compiler_params=pltpu.CompilerParams(use_tc_tiling_on_sc=True),
    )
    def body(idx_ref, zeros_ref, out_ref, zblk, idx_v, pos_v, ones_b, sem_z, sem_s):
        cid = lax.axis_index("c")
        sid = lax.axis_index("s")
        tile_row0 = (cid * NS + sid) * ROWS_PER_T

        @pl.when(sid == 0)
        def _():
            pltpu.sync_copy(zeros_ref, zblk)

        plsc.subcore_barrier()

        sc_base = cid * SC_FLOATS
        for k in range(Z_PER_T):
            dst = out_ref.at[pl.ds(sc_base + (sid * Z_PER_T + k) * Z, Z)]
            pltpu.async_copy(zblk, dst, sem_z)

        pltpu.sync_copy(idx_ref.at[pl.ds(tile_row0, ROWS_PER_T)], idx_v)
        lane = lax.iota(jnp.int32, L)
        ones_v = jnp.full((L,), 1.0, jnp.float32)
        for q in range(K // L):
            ones_b[pl.ds(q * L, L)] = ones_v

        def fill_pos(n, carry):
            for q in range(K // L):
                g = n * K + q * L
                jl = g // 256
                il = g - jl * 256
                it = sid * 2 + il // 128
                sterm = (
                    (cid * 10 + jl) * (N_EMB * 4096)
                    + it * 1024
                    + (il % 128)
                )
                k16 = idx_v[pl.ds(g, L)]
                pos_v[n, pl.ds(q * L, L)] = (
                    sterm + (k16 >> 3) * 32768 + (k16 & 7) * 128 + lane
                )
            return carry

        lax.fori_loop(0, NK, fill_pos, 0)

        for k in range(Z_PER_T):
            pltpu.make_async_copy(zblk, out_ref.at[pl.ds(0, Z)], sem_z).wait()
        plsc.subcore_barrier()

        for j in range(NK):
            pltpu.async_copy(ones_b, out_ref.at[pos_v.at[j]], sem_s)
        for j in range(NK):
            pltpu.make_async_copy(ones_b, out_ref.at[pos_v.at[0]], sem_s).wait()

    return body(idx_hbm, zeros_hbm)


@jax.jit
def kernel(inputs):
    idx_t = inputs.astype(jnp.int32).T
    idx_perm = jnp.transpose(
        idx_t.reshape(NC, 10, NS, 256), (0, 2, 1, 3)
    ).reshape(-1)
    zeros = jnp.zeros((Z,), jnp.float32)
    flat = _one_hot_sc(idx_perm, zeros)
    out5 = flat.reshape(20, N_EMB // 8, 4096 // 128, 8, 128)
    out3 = jnp.transpose(out5, (0, 1, 3, 2, 4)).reshape(20, N_EMB, 4096)
    return jnp.transpose(out3, (2, 0, 1))


# speedup vs baseline: 1.0620x; 1.0444x over previous
import functools

import jax
import jax.numpy as jnp
from jax import lax
from jax.experimental import pallas as pl
from jax.experimental.pallas import tpu as pltpu
from jax.experimental.pallas import tpu_sc as plsc

N_EMB = 1000
ROWS = 4096 * 20
NC, NS, L = 2, 16, 16
ROWS_PER_T = ROWS // (NC * NS)
Z = 640_000
ZT = 64_000
FS = 3
FT = 10
SC_FLOATS = ROWS // NC * N_EMB
FL_PER_T = FS * Z + FT * ZT
K = 128
NK = ROWS_PER_T // K


def _one_hot_sc(idx_hbm, zeros_hbm):
    mesh = plsc.VectorSubcoreMesh(
        core_axis_name="c", subcore_axis_name="s", num_cores=NC, num_subcores=NS
    )

    @functools.partial(
        pl.kernel,
        out_type=jax.ShapeDtypeStruct((ROWS * N_EMB,), jnp.float32),
        mesh=mesh,
        scratch_types=[
            pltpu.VMEM_SHARED((Z,), jnp.float32),
            pltpu.VMEM((ZT,), jnp.float32),
            pltpu.VMEM((ROWS_PER_T,), jnp.int32),
            pltpu.VMEM((NK, K), jnp.int32),
            pltpu.VMEM((K,), jnp.float32),
            pltpu.SemaphoreType.DMA,
            pltpu.SemaphoreType.DMA,
        ],
        compiler_params=pltpu.CompilerParams(use_tc_tiling_on_sc=True),
    )
    def body(idx_ref, zeros_ref, out_ref, zblk, ztile, idx_v, pos_v, ones_b, sem_z, sem_s):
        cid = lax.axis_index("c")
        sid = lax.axis_index("s")
        tile_row0 = (cid * NS + sid) * ROWS_PER_T

        zero16 = jnp.zeros((L,), jnp.float32)

        def zfill(t, carry):
            ztile[pl.ds(t * L, L)] = zero16
            return carry

        lax.fori_loop(0, ZT // L, zfill, 0)

        @pl.when(sid == 0)
        def _():
            pltpu.sync_copy(zeros_ref, zblk)

        plsc.subcore_barrier()

        sc_base = cid * SC_FLOATS
        t_base = sc_base + sid * FL_PER_T
        for k in range(FS):
            dst = out_ref.at[pl.ds(t_base + k * Z, Z)]
            pltpu.async_copy(zblk, dst, sem_z)
        for k in range(FT):
            dst = out_ref.at[pl.ds(t_base + FS * Z + k * ZT, ZT)]
            pltpu.async_copy(ztile, dst, sem_z)

        pltpu.sync_copy(idx_ref.at[pl.ds(tile_row0, ROWS_PER_T)], idx_v)
        lane = lax.iota(jnp.int32, L)
        ones_v = jnp.full((L,), 1.0, jnp.float32)
        for q in range(K // L):
            ones_b[pl.ds(q * L, L)] = ones_v

        def fill_pos(n, carry):
            for q in range(K // L):
                g = n * K + q * L
                jl = g // 256
                il = g - jl * 256
                it = sid * 2 + il // 128
                sterm = (
                    (cid * 10 + jl) * (N_EMB * 4096)
                    + it * 1024
                    + (il % 128)
                )
                k16 = idx_v[pl.ds(g, L)]
                pos_v[n, pl.ds(q * L, L)] = (
                    sterm + (k16 >> 3) * 32768 + (k16 & 7) * 128 + lane
                )
            return carry

        lax.fori_loop(0, NK, fill_pos, 0)

        for k in range(FS):
            pltpu.make_async_copy(zblk, out_ref.at[pl.ds(0, Z)], sem_z).wait()
        for k in range(FT):
            pltpu.make_async_copy(ztile, out_ref.at[pl.ds(0, ZT)], sem_z).wait()
        plsc.subcore_barrier()

        for j in range(NK):
            pltpu.async_copy(ones_b, out_ref.at[pos_v.at[j]], sem_s)
        for j in range(NK):
            pltpu.make_async_copy(ones_b, out_ref.at[pos_v.at[0]], sem_s).wait()

    return body(idx_hbm, zeros_hbm)


@jax.jit
def kernel(inputs):
    idx_t = inputs.astype(jnp.int32).T
    idx_perm = jnp.transpose(
        idx_t.reshape(NC, 10, NS, 256), (0, 2, 1, 3)
    ).reshape(-1)
    zeros = jnp.zeros((Z,), jnp.float32)
    flat = _one_hot_sc(idx_perm, zeros)
    out5 = flat.reshape(20, N_EMB // 8, 4096 // 128, 8, 128)
    out3 = jnp.transpose(out5, (0, 1, 3, 2, 4)).reshape(20, N_EMB, 4096)
    return jnp.transpose(out3, (2, 0, 1))


# speedup vs baseline: 1.2099x; 1.1393x over previous
import functools

import jax
import jax.numpy as jnp
from jax import lax
from jax.experimental import pallas as pl
from jax.experimental.pallas import tpu as pltpu
from jax.experimental.pallas import tpu_sc as plsc

N_EMB = 1000
ROWS = 4096 * 20
NC, NS, L = 2, 16, 16
ROWS_PER_T = ROWS // (NC * NS)
Z = 640_000
ZT = 64_000
FS = 2
FT = 20
SC_FLOATS = ROWS // NC * N_EMB
FL_PER_T = FS * Z + FT * ZT
K = 128
NK = ROWS_PER_T // K


def _one_hot_sc(idx_hbm, zeros_hbm):
    mesh = plsc.VectorSubcoreMesh(
        core_axis_name="c", subcore_axis_name="s", num_cores=NC, num_subcores=NS
    )

    @functools.partial(
        pl.kernel,
        out_type=jax.ShapeDtypeStruct((ROWS * N_EMB,), jnp.float32),
        mesh=mesh,
        scratch_types=[
            pltpu.VMEM_SHARED((Z,), jnp.float32),
            pltpu.VMEM((ZT,), jnp.float32),
            pltpu.VMEM((ROWS_PER_T,), jnp.int32),
            pltpu.VMEM((NK, K), jnp.int32),
            pltpu.VMEM((K,), jnp.float32),
            pltpu.SemaphoreType.DMA,
            pltpu.SemaphoreType.DMA,
        ],
        compiler_params=pltpu.CompilerParams(use_tc_tiling_on_sc=True),
    )
    def body(idx_ref, zeros_ref, out_ref, zblk, ztile, idx_v, pos_v, ones_b, sem_z, sem_s):
        cid = lax.axis_index("c")
        sid = lax.axis_index("s")
        tile_row0 = (cid * NS + sid) * ROWS_PER_T

        zero16 = jnp.zeros((L,), jnp.float32)

        def zfill(t, carry):
            ztile[pl.ds(t * L, L)] = zero16
            return carry

        lax.fori_loop(0, ZT // L, zfill, 0)

        @pl.when(sid == 0)
        def _():
            pltpu.sync_copy(zeros_ref, zblk)

        plsc.subcore_barrier()

        sc_base = cid * SC_FLOATS
        t_base = sc_base + sid * FL_PER_T
        for k in range(FS):
            dst = out_ref.at[pl.ds(t_base + k * Z, Z)]
            pltpu.async_copy(zblk, dst, sem_z)
        for k in range(FT):
            dst = out_ref.at[pl.ds(t_base + FS * Z + k * ZT, ZT)]
            pltpu.async_copy(ztile, dst, sem_z)

        pltpu.sync_copy(idx_ref.at[pl.ds(tile_row0, ROWS_PER_T)], idx_v)
        lane = lax.iota(jnp.int32, L)
        ones_v = jnp.full((L,), 1.0, jnp.float32)
        for q in range(K // L):
            ones_b[pl.ds(q * L, L)] = ones_v

        def fill_pos(n, carry):
            for q in range(K // L):
                g = n * K + q * L
                jl = g // 256
                il = g - jl * 256
                it = sid * 2 + il // 128
                sterm = (
                    (cid * 10 + jl) * (N_EMB * 4096)
                    + it * 1024
                    + (il % 128)
                )
                k16 = idx_v[pl.ds(g, L)]
                pos_v[n, pl.ds(q * L, L)] = (
                    sterm + (k16 >> 3) * 32768 + (k16 & 7) * 128 + lane
                )
            return carry

        lax.fori_loop(0, NK, fill_pos, 0)

        for k in range(FS):
            pltpu.make_async_copy(zblk, out_ref.at[pl.ds(0, Z)], sem_z).wait()
        for k in range(FT):
            pltpu.make_async_copy(ztile, out_ref.at[pl.ds(0, ZT)], sem_z).wait()
        plsc.subcore_barrier()

        for j in range(NK):
            pltpu.async_copy(ones_b, out_ref.at[pos_v.at[j]], sem_s)
        for j in range(NK):
            pltpu.make_async_copy(ones_b, out_ref.at[pos_v.at[0]], sem_s).wait()

    return body(idx_hbm, zeros_hbm)


@jax.jit
def kernel(inputs):
    idx_t = inputs.astype(jnp.int32).T
    idx_perm = jnp.transpose(
        idx_t.reshape(NC, 10, NS, 256), (0, 2, 1, 3)
    ).reshape(-1)
    zeros = jnp.zeros((Z,), jnp.float32)
    flat = _one_hot_sc(idx_perm, zeros)
    out5 = flat.reshape(20, N_EMB // 8, 4096 // 128, 8, 128)
    out3 = jnp.transpose(out5, (0, 1, 3, 2, 4)).reshape(20, N_EMB, 4096)
    return jnp.transpose(out3, (2, 0, 1))


# speedup vs baseline: 1.2338x; 1.0198x over previous
import functools

import jax
import jax.numpy as jnp
from jax import lax
from jax.experimental import pallas as pl
from jax.experimental.pallas import tpu as pltpu
from jax.experimental.pallas import tpu_sc as plsc

N_EMB = 1000
ROWS = 4096 * 20
NC, NS, L = 2, 16, 16
ROWS_PER_T = ROWS // (NC * NS)
Z = 640_000
ZT = 64_000
FS = 1
FT = 30
SC_FLOATS = ROWS // NC * N_EMB
FL_PER_T = FS * Z + FT * ZT
K = 128
NK = ROWS_PER_T // K


def _one_hot_sc(idx_hbm, zeros_hbm):
    mesh = plsc.VectorSubcoreMesh(
        core_axis_name="c", subcore_axis_name="s", num_cores=NC, num_subcores=NS
    )

    @functools.partial(
        pl.kernel,
        out_type=jax.ShapeDtypeStruct((ROWS * N_EMB,), jnp.float32),
        mesh=mesh,
        scratch_types=[
            pltpu.VMEM_SHARED((Z,), jnp.float32),
            pltpu.VMEM((ZT,), jnp.float32),
            pltpu.VMEM((ROWS_PER_T,), jnp.int32),
            pltpu.VMEM((NK, K), jnp.int32),
            pltpu.VMEM((K,), jnp.float32),
            pltpu.SemaphoreType.DMA,
            pltpu.SemaphoreType.DMA,
        ],
        compiler_params=pltpu.CompilerParams(use_tc_tiling_on_sc=True),
    )
    def body(idx_ref, zeros_ref, out_ref, zblk, ztile, idx_v, pos_v, ones_b, sem_z, sem_s):
        cid = lax.axis_index("c")
        sid = lax.axis_index("s")
        tile_row0 = (cid * NS + sid) * ROWS_PER_T

        zero16 = jnp.zeros((L,), jnp.float32)

        def zfill(t, carry):
            ztile[pl.ds(t * L, L)] = zero16
            return carry

        lax.fori_loop(0, ZT // L, zfill, 0)

        @pl.when(sid == 0)
        def _():
            pltpu.sync_copy(zeros_ref, zblk)

        plsc.subcore_barrier()

        sc_base = cid * SC_FLOATS
        t_base = sc_base + sid * FL_PER_T
        for k in range(FS):
            dst = out_ref.at[pl.ds(t_base + k * Z, Z)]
            pltpu.async_copy(zblk, dst, sem_z)
        for k in range(FT):
            dst = out_ref.at[pl.ds(t_base + FS * Z + k * ZT, ZT)]
            pltpu.async_copy(ztile, dst, sem_z)

        pltpu.sync_copy(idx_ref.at[pl.ds(tile_row0, ROWS_PER_T)], idx_v)
        lane = lax.iota(jnp.int32, L)
        ones_v = jnp.full((L,), 1.0, jnp.float32)
        for q in range(K // L):
            ones_b[pl.ds(q * L, L)] = ones_v

        def fill_pos(n, carry):
            for q in range(K // L):
                g = n * K + q * L
                jl = g // 256
                il = g - jl * 256
                it = sid * 2 + il // 128
                sterm = (
                    (cid * 10 + jl) * (N_EMB * 4096)
                    + it * 1024
                    + (il % 128)
                )
                k16 = idx_v[pl.ds(g, L)]
                pos_v[n, pl.ds(q * L, L)] = (
                    sterm + (k16 >> 3) * 32768 + (k16 & 7) * 128 + lane
                )
            return carry

        lax.fori_loop(0, NK, fill_pos, 0)

        for k in range(FS):
            pltpu.make_async_copy(zblk, out_ref.at[pl.ds(0, Z)], sem_z).wait()
        for k in range(FT):
            pltpu.make_async_copy(ztile, out_ref.at[pl.ds(0, ZT)], sem_z).wait()
        plsc.subcore_barrier()

        for j in range(NK):
            pltpu.async_copy(ones_b, out_ref.at[pos_v.at[j]], sem_s)
        for j in range(NK):
            pltpu.make_async_copy(ones_b, out_ref.at[pos_v.at[0]], sem_s).wait()

    return body(idx_hbm, zeros_hbm)


@jax.jit
def kernel(inputs):
    idx_t = inputs.astype(jnp.int32).T
    idx_perm = jnp.transpose(
        idx_t.reshape(NC, 10, NS, 256), (0, 2, 1, 3)
    ).reshape(-1)
    zeros = jnp.zeros((Z,), jnp.float32)
    flat = _one_hot_sc(idx_perm, zeros)
    out5 = flat.reshape(20, N_EMB // 8, 4096 // 128, 8, 128)
    out3 = jnp.transpose(out5, (0, 1, 3, 2, 4)).reshape(20, N_EMB, 4096)
    return jnp.transpose(out3, (2, 0, 1))
